# Initial kernel scaffold; baseline (speedup 1.0000x reference)
import jax
import jax.numpy as jnp
from jax.experimental import pallas as pl


def kernel(features, edge_index, ff_W1, ff_b1, ff_W2, ff_b2, ffln_g, ffln_b, gcn_W, gcn_b, ln_g, ln_b):
    raise NotImplementedError("write your pallas kernel here")


# speedup vs baseline: 4.5563x; 4.5563x over previous
import functools

import jax
import jax.numpy as jnp
from jax import lax
from jax.experimental import pallas as pl
from jax.experimental.pallas import tpu as pltpu
from jax.experimental.pallas import tpu_sc as plsc

N = 10000
E = 320000
D = 128
EPS = 1e-6

NC = 2
NS = 16
NW = NC * NS
EPT = E // NW
CHUNK = 80
NCHUNK = EPT // CHUNK
ACC_ROWS = 10240
RPT = ACC_ROWS // NS

BLK = 80


def _layer_norm_block(t, g, b):
    mu = jnp.mean(t, axis=-1, keepdims=True)
    var = jnp.mean((t - mu) ** 2, axis=-1, keepdims=True)
    return g * (t - mu) * lax.rsqrt(var + EPS) + b


def _ffn_body(x_ref, w1_ref, b1_ref, w2_ref, b2_ref, g_ref, b_ref, o_ref):
    x = x_ref[...]
    h = jnp.maximum(
        jnp.dot(x, w1_ref[...], preferred_element_type=jnp.float32) + b1_ref[...], 0.0)
    f = jnp.dot(h, w2_ref[...], preferred_element_type=jnp.float32) + b2_ref[...]
    o_ref[...] = _layer_norm_block(f + x, g_ref[...], b_ref[...])


def _ffn_ln(x, w1, b1, w2, b2, g, b):
    nblk = N // BLK
    row = pl.BlockSpec((BLK, D), lambda i: (i, 0))
    full = pl.BlockSpec((D, D), lambda i: (0, 0))
    vec = pl.BlockSpec((1, D), lambda i: (0, 0))
    return pl.pallas_call(
        _ffn_body,
        grid=(nblk,),
        in_specs=[row, full, vec, full, vec, vec, vec],
        out_specs=row,
        out_shape=jax.ShapeDtypeStruct((N, D), jnp.float32),
    )(x, w1, b1, w2, b2, g, b)


def _gcn_body(p0_ref, p1_ref, ff_ref, w_ref, b_ref, g_ref, bb_ref, o_ref):
    agg = p0_ref[...] + p1_ref[...]
    attn = jnp.maximum(
        jnp.dot(agg, w_ref[...], preferred_element_type=jnp.float32) + b_ref[...], 0.0)
    o_ref[...] = _layer_norm_block(attn + ff_ref[...], g_ref[...], bb_ref[...])


def _gcn_ln(partials, ff_out, w, b, g, bb):
    nblk = N // BLK
    off = ACC_ROWS // BLK
    p0 = pl.BlockSpec((BLK, D), lambda i: (i, 0))
    p1 = pl.BlockSpec((BLK, D), lambda i: (i + off, 0))
    row = pl.BlockSpec((BLK, D), lambda i: (i, 0))
    full = pl.BlockSpec((D, D), lambda i: (0, 0))
    vec = pl.BlockSpec((1, D), lambda i: (0, 0))
    return pl.pallas_call(
        _gcn_body,
        grid=(nblk,),
        in_specs=[p0, p1, row, full, vec, vec, vec],
        out_specs=row,
        out_shape=jax.ShapeDtypeStruct((N, D), jnp.float32),
    )(partials, partials, ff_out, w, b, g, bb)


def _sc_body(ff_hbm, src_hbm, dst_hbm, out_hbm, src_v, dst_v, rows_v, zbuf, acc, sem):
    cid = lax.axis_index("c")
    sid = lax.axis_index("s")
    wid = sid * NC + cid

    zero = jnp.zeros((16,), jnp.float32)
    for r in range(16):
        for c in range(D // 16):
            zbuf[r, pl.ds(c * 16, 16)] = zero
    row0 = sid * RPT

    def zero_rows(i, carry):
        pltpu.sync_copy(zbuf, acc.at[pl.ds(row0 + i * 16, 16)])
        return carry

    lax.fori_loop(0, RPT // 16, zero_rows, 0)

    pltpu.sync_copy(src_hbm.at[wid], src_v)
    pltpu.sync_copy(dst_hbm.at[wid], dst_v)
    plsc.subcore_barrier()

    def body(j, carry):
        pltpu.async_copy(ff_hbm.at[src_v.at[j]], rows_v, sem).wait()
        pltpu.sync_copy(rows_v, acc.at[dst_v.at[j]], add=True)
        return carry

    lax.fori_loop(0, NCHUNK, body, 0)
    plsc.subcore_barrier()

    out_base = cid * ACC_ROWS + row0
    pltpu.sync_copy(acc.at[pl.ds(row0, RPT)], out_hbm.at[pl.ds(out_base, RPT)])


def _sc_segment_sum(ff_out, src_r, dst_r):
    mesh = plsc.VectorSubcoreMesh(core_axis_name="c", subcore_axis_name="s")
    k = functools.partial(
        pl.kernel,
        out_type=jax.ShapeDtypeStruct((NC * ACC_ROWS, D), jnp.float32),
        mesh=mesh,
        scratch_types=[
            pltpu.VMEM((NCHUNK, CHUNK), jnp.int32),
            pltpu.VMEM((NCHUNK, CHUNK), jnp.int32),
            pltpu.VMEM((CHUNK, D), jnp.float32),
            pltpu.VMEM((16, D), jnp.float32),
            pltpu.VMEM_SHARED((ACC_ROWS, D), jnp.float32),
            pltpu.SemaphoreType.DMA,
        ],
    )(_sc_body)
    return k(ff_out, src_r, dst_r)


def kernel(features, edge_index, ff_W1, ff_b1, ff_W2, ff_b2, ffln_g, ffln_b,
           gcn_W, gcn_b, ln_g, ln_b):
    src_r = edge_index[0].reshape(NW, NCHUNK, CHUNK)
    dst_r = edge_index[1].reshape(NW, NCHUNK, CHUNK)
    output = features
    L = ff_W1.shape[0]
    for i in range(L):
        ff_out = _ffn_ln(output, ff_W1[i], ff_b1[i].reshape(1, D),
                         ff_W2[i], ff_b2[i].reshape(1, D),
                         ffln_g[i].reshape(1, D), ffln_b[i].reshape(1, D))
        partials = _sc_segment_sum(ff_out, src_r, dst_r)
        output = _gcn_ln(partials, ff_out, gcn_W[i], gcn_b[i].reshape(1, D),
                         ln_g[i].reshape(1, D), ln_b[i].reshape(1, D))
    return output
